# baseline (speedup 1.0000x reference)
import jax
import jax.numpy as jnp
from jax import lax
from jax.experimental import pallas as pl
from jax.experimental.pallas import tpu as pltpu
from jax.experimental.pallas import tpu_sc as plsc

NC = 2
NS = 16
NW = NC * NS
W = 8
NBUF = 2


def _make_spmm(n_pad, n_edges, chunk):
  span = n_pad // NS
  q = span // 8
  per_w = n_edges // NW
  n_loop = per_w // (chunk * NBUF)
  assert per_w % (chunk * NBUF) == 0 and chunk % 8 == 0
  mesh = plsc.VectorSubcoreMesh(
      core_axis_name="c", subcore_axis_name="s",
      num_cores=NC, num_subcores=NS)

  out_type = jax.ShapeDtypeStruct((NC * n_pad, W), jnp.float32)
  scratch = (
      [pltpu.VMEM_SHARED((n_pad, W), jnp.float32)] * 2
      + [pltpu.VMEM((chunk,), jnp.int32)] * (2 * NBUF)
      + [pltpu.VMEM((chunk, W), jnp.float32)] * NBUF
      + [pltpu.VMEM((q, W), jnp.float32)]
      + [pltpu.SemaphoreType.DMA] * (4 * NBUF)
  )

  def body(table_hbm, zeros_hbm, row_hbm, col_hbm, acc_out, *rest):
    ev_sh, acc_sh = rest[0], rest[1]
    ridx = rest[2:2 + NBUF]
    cidx = rest[2 + NBUF:2 + 2 * NBUF]
    vp = rest[2 + 2 * NBUF:2 + 3 * NBUF]
    stage = rest[2 + 3 * NBUF]
    sems = rest[3 + 3 * NBUF:]
    sem_row = sems[0:NBUF]
    sem_col = sems[NBUF:2 * NBUF]
    sem_g = sems[2 * NBUF:3 * NBUF]
    sem_s = sems[3 * NBUF:4 * NBUF]

    c = lax.axis_index("c")
    s = lax.axis_index("s")
    off = s * span
    for k in range(8):
      pltpu.sync_copy(table_hbm.at[pl.ds(off + k * q, q)], stage)
      pltpu.sync_copy(stage, ev_sh.at[pl.ds(off + k * q, q)])
    pltpu.sync_copy(zeros_hbm.at[pl.ds(0, q)], stage)
    for k in range(8):
      pltpu.sync_copy(stage, acc_sh.at[pl.ds(off + k * q, q)])
    plsc.subcore_barrier()

    base = (c * NS + s) * per_w

    def inner(j, carry):
      o_base = base + j * (chunk * NBUF)
      din = []
      for b in range(NBUF):
        o = o_base + b * chunk
        din.append((
            pltpu.async_copy(row_hbm.at[pl.ds(o, chunk)], ridx[b],
                             sem_row[b]),
            pltpu.async_copy(col_hbm.at[pl.ds(o, chunk)], cidx[b],
                             sem_col[b])))
      dsc = []
      for b in range(NBUF):
        dr, dc = din[b]
        dr.wait()
        dc.wait()
        pltpu.async_copy(ev_sh.at[ridx[b]], vp[b], sem_g[b]).wait()
        dsc.append(pltpu.async_copy(vp[b], acc_sh.at[cidx[b]],
                                    sem_s[b], add=True))
      for d in dsc:
        d.wait()
      return carry

    lax.fori_loop(0, n_loop, inner, 0)
    plsc.subcore_barrier()

    for k in range(8):
      o = off + k * q
      pltpu.sync_copy(acc_sh.at[pl.ds(o, q)], stage)
      pltpu.sync_copy(stage, acc_out.at[pl.ds(c * n_pad + o, q)])

  return pl.kernel(
      body, out_type=out_type, mesh=mesh, scratch_types=tuple(scratch),
      compiler_params=pltpu.CompilerParams(use_tc_tiling_on_sc=False))


def kernel(e, edge_index, prop_layers=2, alpha=0.5):
  n = e.shape[0]
  n_edges = edge_index.shape[1]
  n_pad = -(-n // (NS * 64)) * (NS * 64)
  chunk = 800
  assert n_edges % (NW * chunk * NBUF) == 0

  row = edge_index[0].astype(jnp.int32)
  col = edge_index[1].astype(jnp.int32)
  e_pad = jnp.zeros((n_pad,), jnp.float32).at[:n].set(e.astype(jnp.float32))
  zeros8 = jnp.zeros((n_pad, W), jnp.float32)

  spmm = _make_spmm(n_pad, n_edges, chunk)

  def table_of(v):
    return zeros8.at[:, 0].set(v).at[:, 1].set(1.0)

  acc1_p = spmm(table_of(e_pad), zeros8, row, col)
  acc1 = acc1_p[:n_pad, 0] + acc1_p[n_pad:, 0]
  deg = acc1_p[:n_pad, 1] + acc1_p[n_pad:, 1]
  inv_deg = jnp.where(deg > 0, 1.0 / deg, 0.0)

  a = jnp.float32(alpha)
  ev1 = a * e_pad + (1.0 - a) * inv_deg * acc1
  acc2_p = spmm(table_of(ev1), zeros8, row, col)
  acc2 = acc2_p[:n_pad, 0] + acc2_p[n_pad:, 0]
  ev2 = a * ev1 + (1.0 - a) * inv_deg * acc2
  return ev2[:n]


# speedup vs baseline: 2.7730x; 2.7730x over previous
import jax
import jax.numpy as jnp
from jax import lax
from jax.experimental import pallas as pl
from jax.experimental.pallas import tpu as pltpu
from jax.experimental.pallas import tpu_sc as plsc

NC = 2
NS = 16
NW = NC * NS
L = 16


def _mesh():
  return plsc.VectorSubcoreMesh(
      core_axis_name="c", subcore_axis_name="s",
      num_cores=NC, num_subcores=NS)


def _pipeline(row_hbm, col_hbm, ev_sh, ridx, cidx, vals, sems, base, per_w,
              chunk, scatters):
  n_chunks = per_w // chunk
  sem_row, sem_col, sem_g = sems[0:3], sems[3:6], sems[6:9]

  def start_in(i, b):
    o = base + i * chunk
    return (pltpu.async_copy(row_hbm.at[pl.ds(o, chunk)], ridx[b],
                             sem_row[b]),
            pltpu.async_copy(col_hbm.at[pl.ds(o, chunk)], cidx[b],
                             sem_col[b]))

  pend_in = {0: start_in(0, 0)}
  pend_sc = {}
  for i in range(n_chunks):
    b = i % 3
    if i - 2 in pend_sc:
      for d in pend_sc.pop(i - 2):
        d.wait()
    if i + 1 < n_chunks:
      pend_in[i + 1] = start_in(i + 1, (i + 1) % 3)
    dr, dc = pend_in.pop(i)
    dr.wait()
    dc.wait()
    pltpu.async_copy(ev_sh.at[ridx[b]], vals[b], sem_g[b]).wait()
    pend_sc[i] = scatters(b)
  for descs in pend_sc.values():
    for d in descs:
      d.wait()


def _make_pass1(n_pad, n_edges, chunk):
  span = n_pad // NS
  per_w = n_edges // NW
  out_type = (jax.ShapeDtypeStruct((NC * n_pad,), jnp.float32),
              jax.ShapeDtypeStruct((NC * n_pad,), jnp.float32))
  scratch = (
      [pltpu.VMEM_SHARED((n_pad,), jnp.float32)] * 3
      + [pltpu.VMEM((chunk,), jnp.int32)] * 6
      + [pltpu.VMEM((chunk,), jnp.float32)] * 4
      + [pltpu.VMEM((span,), jnp.float32)]
      + [pltpu.SemaphoreType.DMA] * 15
  )

  def body(e_hbm, zeros_hbm, ones_hbm, row_hbm, col_hbm,
           acc_out, deg_out, *rest):
    ev_sh, acc_sh, deg_sh = rest[0:3]
    ridx, cidx = rest[3:6], rest[6:9]
    vals, ones_v = rest[9:12], rest[12]
    stage = rest[13]
    sems = rest[14:]
    sem_s, sem_d = sems[9:12], sems[12:15]

    c = lax.axis_index("c")
    s = lax.axis_index("s")
    off = s * span
    pltpu.sync_copy(e_hbm.at[pl.ds(off, span)], stage)
    pltpu.sync_copy(stage, ev_sh.at[pl.ds(off, span)])
    pltpu.sync_copy(zeros_hbm.at[pl.ds(off, span)], stage)
    pltpu.sync_copy(stage, acc_sh.at[pl.ds(off, span)])
    pltpu.sync_copy(stage, deg_sh.at[pl.ds(off, span)])
    pltpu.sync_copy(ones_hbm, ones_v)
    plsc.subcore_barrier()

    base = (c * NS + s) * per_w

    def scatters(b):
      return (
          pltpu.async_copy(vals[b], acc_sh.at[cidx[b]], sem_s[b], add=True),
          pltpu.async_copy(ones_v, deg_sh.at[cidx[b]], sem_d[b], add=True))

    _pipeline(row_hbm, col_hbm, ev_sh, ridx, cidx, vals, sems, base, per_w,
              chunk, scatters)
    plsc.subcore_barrier()

    oo = c * n_pad + off
    pltpu.sync_copy(acc_sh.at[pl.ds(off, span)], stage)
    pltpu.sync_copy(stage, acc_out.at[pl.ds(oo, span)])
    pltpu.sync_copy(deg_sh.at[pl.ds(off, span)], stage)
    pltpu.sync_copy(stage, deg_out.at[pl.ds(oo, span)])

  return pl.kernel(body, out_type=out_type, mesh=_mesh(),
                   scratch_types=tuple(scratch))


def _make_pass2(n_pad, n_edges, chunk):
  span = n_pad // NS
  per_w = n_edges // NW
  out_type = jax.ShapeDtypeStruct((NC * n_pad,), jnp.float32)
  scratch = (
      [pltpu.VMEM_SHARED((n_pad,), jnp.float32)] * 2
      + [pltpu.VMEM((chunk,), jnp.int32)] * 6
      + [pltpu.VMEM((chunk,), jnp.float32)] * 3
      + [pltpu.VMEM((span,), jnp.float32)] * 4
      + [pltpu.VMEM((L,), jnp.float32)]
      + [pltpu.SemaphoreType.DMA] * 12
  )

  def body(e_hbm, accp_hbm, degp_hbm, alpha_hbm, zeros_hbm, row_hbm, col_hbm,
           outp, *rest):
    ev_sh, acc_sh = rest[0:2]
    ridx, cidx = rest[2:5], rest[5:8]
    vals = rest[8:11]
    bufA, bufB, inv_v, ev1_v = rest[11:15]
    alpha_v = rest[15]
    sems = rest[16:]
    sem_s = sems[9:12]

    c = lax.axis_index("c")
    s = lax.axis_index("s")
    off = s * span

    pltpu.sync_copy(alpha_hbm, alpha_v)
    va = alpha_v[...]
    om = 1.0 - va
    ha = va * 0.5

    def vloop(f):
      def step(k, carry):
        f(pl.ds(k * L, L))
        return carry
      lax.fori_loop(0, span // L, step, 0)

    pltpu.sync_copy(degp_hbm.at[pl.ds(off, span)], inv_v)
    pltpu.sync_copy(degp_hbm.at[pl.ds(n_pad + off, span)], bufA)

    def f_deg(sl):
      deg = inv_v[sl] + bufA[sl]
      inv_v[sl] = jnp.where(deg > 0.0, 1.0 / deg, 0.0)
    vloop(f_deg)

    pltpu.sync_copy(accp_hbm.at[pl.ds(off, span)], bufA)
    pltpu.sync_copy(accp_hbm.at[pl.ds(n_pad + off, span)], bufB)

    def f_acc(sl):
      bufA[sl] = bufA[sl] + bufB[sl]
    vloop(f_acc)

    pltpu.sync_copy(e_hbm.at[pl.ds(off, span)], bufB)

    def f_ev1(sl):
      ev1_v[sl] = va * bufB[sl] + om * inv_v[sl] * bufA[sl]
    vloop(f_ev1)

    pltpu.sync_copy(ev1_v, ev_sh.at[pl.ds(off, span)])
    pltpu.sync_copy(zeros_hbm.at[pl.ds(off, span)], bufA)
    pltpu.sync_copy(bufA, acc_sh.at[pl.ds(off, span)])
    plsc.subcore_barrier()

    base = (c * NS + s) * per_w

    def scatters(b):
      return (
          pltpu.async_copy(vals[b], acc_sh.at[cidx[b]], sem_s[b], add=True),)

    _pipeline(row_hbm, col_hbm, ev_sh, ridx, cidx, vals, sems, base, per_w,
              chunk, scatters)
    plsc.subcore_barrier()

    pltpu.sync_copy(acc_sh.at[pl.ds(off, span)], bufA)

    def f_out(sl):
      bufB[sl] = ha * ev1_v[sl] + om * inv_v[sl] * bufA[sl]
    vloop(f_out)

    pltpu.sync_copy(bufB, outp.at[pl.ds(c * n_pad + off, span)])

  return pl.kernel(body, out_type=out_type, mesh=_mesh(),
                   scratch_types=tuple(scratch))


def kernel(e, edge_index, prop_layers=2, alpha=0.5):
  n = e.shape[0]
  n_edges = edge_index.shape[1]
  n_pad = -(-n // (NS * 8)) * (NS * 8)
  chunk = 10000
  assert n_edges % (NW * chunk) == 0

  row = edge_index[0].astype(jnp.int32)
  col = edge_index[1].astype(jnp.int32)
  e_pad = jnp.zeros((n_pad,), jnp.float32).at[:n].set(e.astype(jnp.float32))
  zeros = jnp.zeros((n_pad,), jnp.float32)
  ones = jnp.ones((chunk,), jnp.float32)
  a = jnp.float32(alpha)
  alpha_arr = jnp.full((L,), a, jnp.float32)

  accp, degp = _make_pass1(n_pad, n_edges, chunk)(
      e_pad, zeros, ones, row, col)
  outp = _make_pass2(n_pad, n_edges, chunk)(
      e_pad, accp, degp, alpha_arr, zeros, row, col)
  return (outp[:n_pad] + outp[n_pad:])[:n]


# speedup vs baseline: 2.8492x; 1.0275x over previous
import jax
import jax.numpy as jnp
from jax import lax
from jax.experimental import pallas as pl
from jax.experimental.pallas import tpu as pltpu
from jax.experimental.pallas import tpu_sc as plsc

NC = 2
NS = 16
NW = NC * NS


def _make_spmm(n_pad, n_edges, chunk, with_deg):
  span = n_pad // NS
  per_w = n_edges // NW
  n_chunks = per_w // chunk
  mesh = plsc.VectorSubcoreMesh(
      core_axis_name="c", subcore_axis_name="s",
      num_cores=NC, num_subcores=NS)

  out_type = [jax.ShapeDtypeStruct((NC * n_pad,), jnp.float32)]
  scratch = (
      [pltpu.VMEM_SHARED((n_pad,), jnp.float32)] * 2
      + [pltpu.VMEM((chunk,), jnp.int32)] * 6
      + [pltpu.VMEM((chunk,), jnp.float32)] * 3
      + [pltpu.VMEM((span,), jnp.float32)]
      + [pltpu.SemaphoreType.DMA] * 15
  )
  if with_deg:
    out_type.append(jax.ShapeDtypeStruct((NC * n_pad,), jnp.float32))
    scratch += [
        pltpu.VMEM_SHARED((n_pad,), jnp.float32),
        pltpu.VMEM((chunk,), jnp.float32),
    ]

  def body(*refs):
    if with_deg:
      (ev_hbm, zeros_hbm, ones_hbm, row_hbm, col_hbm,
       acc_out, deg_out, ev_sh, acc_sh, *rest) = refs
      rest, deg_sh, ones_v = rest[:-2], rest[-2], rest[-1]
    else:
      (ev_hbm, zeros_hbm, row_hbm, col_hbm,
       acc_out, ev_sh, acc_sh, *rest) = refs
    ridx = rest[0:3]
    cidx = rest[3:6]
    vals = rest[6:9]
    stage = rest[9]
    sems = rest[10:]
    sem_row = sems[0:3]
    sem_col = sems[3:6]
    sem_g = sems[6:9]
    sem_s = sems[9:12]
    sem_d = sems[12:15]

    c = lax.axis_index("c")
    s = lax.axis_index("s")
    off = s * span
    pltpu.sync_copy(ev_hbm.at[pl.ds(off, span)], stage)
    pltpu.sync_copy(stage, ev_sh.at[pl.ds(off, span)])
    pltpu.sync_copy(zeros_hbm.at[pl.ds(off, span)], stage)
    pltpu.sync_copy(stage, acc_sh.at[pl.ds(off, span)])
    if with_deg:
      pltpu.sync_copy(stage, deg_sh.at[pl.ds(off, span)])
      pltpu.sync_copy(ones_hbm, ones_v)
    plsc.subcore_barrier()

    base = (c * NS + s) * per_w

    def start_in(i, b):
      o = base + i * chunk
      dr = pltpu.async_copy(row_hbm.at[pl.ds(o, chunk)], ridx[b], sem_row[b])
      dc = pltpu.async_copy(col_hbm.at[pl.ds(o, chunk)], cidx[b], sem_col[b])
      return dr, dc

    pend_in = {0: start_in(0, 0)}
    pend_sc = {}
    for i in range(n_chunks):
      b = i % 3
      if i - 2 in pend_sc:
        for d in pend_sc.pop(i - 2):
          d.wait()
      if i + 1 < n_chunks:
        pend_in[i + 1] = start_in(i + 1, (i + 1) % 3)
      dr, dc = pend_in.pop(i)
      dr.wait()
      dc.wait()
      pltpu.async_copy(ev_sh.at[ridx[b]], vals[b], sem_g[b]).wait()
      sc = [pltpu.async_copy(vals[b], acc_sh.at[cidx[b]], sem_s[b], add=True)]
      if with_deg:
        sc.append(pltpu.async_copy(ones_v, deg_sh.at[cidx[b]],
                                   sem_d[b], add=True))
      pend_sc[i] = sc
    for descs in pend_sc.values():
      for d in descs:
        d.wait()
    plsc.subcore_barrier()

    oo = c * n_pad + off
    pltpu.sync_copy(acc_sh.at[pl.ds(off, span)], stage)
    pltpu.sync_copy(stage, acc_out.at[pl.ds(oo, span)])
    if with_deg:
      pltpu.sync_copy(deg_sh.at[pl.ds(off, span)], stage)
      pltpu.sync_copy(stage, deg_out.at[pl.ds(oo, span)])

  return pl.kernel(body, out_type=tuple(out_type), mesh=mesh,
                   scratch_types=tuple(scratch))


def kernel(e, edge_index, prop_layers=2, alpha=0.5):
  n = e.shape[0]
  n_edges = edge_index.shape[1]
  n_pad = -(-n // (NS * 8)) * (NS * 8)
  chunk = 10000
  assert n_edges % (NW * chunk) == 0

  row = edge_index[0].astype(jnp.int32)
  col = edge_index[1].astype(jnp.int32)
  e_pad = jnp.zeros((n_pad,), jnp.float32).at[:n].set(e.astype(jnp.float32))
  zeros = jnp.zeros((n_pad,), jnp.float32)
  ones = jnp.ones((chunk,), jnp.float32)

  spmm_deg = _make_spmm(n_pad, n_edges, chunk, with_deg=True)
  spmm = _make_spmm(n_pad, n_edges, chunk, with_deg=False)

  acc1_p, deg_p = spmm_deg(e_pad, zeros, ones, row, col)
  acc1 = acc1_p[:n_pad] + acc1_p[n_pad:]
  deg = deg_p[:n_pad] + deg_p[n_pad:]
  inv_deg = jnp.where(deg > 0, 1.0 / deg, 0.0)

  a = jnp.float32(alpha)
  ev1 = a * e_pad + (1.0 - a) * inv_deg * acc1
  acc2_p = spmm(ev1, zeros, row, col)
  if isinstance(acc2_p, (tuple, list)):
    acc2_p = acc2_p[0]
  acc2 = acc2_p[:n_pad] + acc2_p[n_pad:]
  ev2 = a * ev1 + (1.0 - a) * inv_deg * acc2
  return ev2[:n]
